# baseline (speedup 1.0000x reference)
import functools
import math

import jax
import jax.numpy as jnp
from jax import lax
from jax.experimental import pallas as pl
from jax.experimental.pallas import tpu as pltpu
from jax.experimental.pallas import tpu_sc as plsc
from jax._src.pallas import mpmd as _mpmd

B = 16384
D = 64
V = 1000000
H = 1024
BETA = 0.95

NC = 2
NS = 16
NW = NC * NS
L = 16

BPT = B // NW
KD = D // L

_mesh = plsc.VectorSubcoreMesh(core_axis_name="c", subcore_axis_name="s")
_sc_params = pltpu.CompilerParams(use_tc_tiling_on_sc=False,
                                 needs_layout_passes=False)


def _fill_widx(widx, idx_v):
    jbase = [(lax.iota(jnp.int32, L) + k * L) * V for k in range(KD)]

    def group(g, carry):
        iv = idx_v[pl.ds(g * L, L)]
        for l in range(L):
            c = iv[l]
            off = (g * L + l) * D
            for k in range(KD):
                widx[pl.ds(off + k * L, L)] = jbase[k] + c
        return carry

    lax.fori_loop(0, BPT // L, group, 0)


@functools.partial(
    pl.kernel,
    out_type=(jax.ShapeDtypeStruct((B * D,), jnp.float32),
              jax.ShapeDtypeStruct((B * D,), jnp.float32)),
    mesh=_mesh,
    compiler_params=_sc_params,
    scratch_types=[
        pltpu.VMEM((BPT,), jnp.int32),
        pltpu.VMEM((BPT * D,), jnp.int32),
        pltpu.VMEM((BPT * D,), jnp.float32),
        pltpu.VMEM((BPT * D,), jnp.float32),
        pltpu.SemaphoreType.DMA,
        pltpu.SemaphoreType.DMA,
    ],
)
def _sc_gather(ind_hbm, aux_hbm, proto_hbm, v_out, z_out, idx_v, widx,
               ra, rb, s1, s2):
    wid = lax.axis_index("s") * NC + lax.axis_index("c")
    base = wid * BPT
    pltpu.sync_copy(ind_hbm.at[pl.ds(base, BPT)], idx_v)
    _fill_widx(widx, idx_v)
    ca = pltpu.async_copy(aux_hbm.at[widx], ra, s1)
    cb = pltpu.async_copy(proto_hbm.at[widx], rb, s2)
    ca.wait()
    cb.wait()
    pltpu.sync_copy(ra, v_out.at[pl.ds(base * D, BPT * D)])
    pltpu.sync_copy(rb, z_out.at[pl.ds(base * D, BPT * D)])


def _vrsqrt(s):
    i = plsc.bitcast(s, jnp.int32)
    y = plsc.bitcast(jnp.int32(0x5F3759DF) - (i >> 1), jnp.float32)
    for _ in range(4):
        y = y * (1.5 - 0.5 * s * y * y)
    return y


def _upd_body(ind_hbm, z_hbm, proto_hbm, out_hbm, idx_v, widx, rows, ssem):
    wid = lax.axis_index("s") * NC + lax.axis_index("c")
    base = wid * BPT
    pltpu.sync_copy(ind_hbm.at[pl.ds(base, BPT)], idx_v)
    _fill_widx(widx, idx_v)
    pltpu.sync_copy(z_hbm.at[pl.ds(base * D, BPT * D)], rows)

    def group(g, carry):
        for r in range(L):
            off = (g * L + r) * D
            regs = [rows[pl.ds(off + k * L, L)] for k in range(KD)]
            t = regs[0] * regs[0]
            for q in regs[1:]:
                t = t + q * q
            s = jnp.maximum(jnp.sum(t), 1e-24)
            sv = jnp.full((L,), s, dtype=jnp.float32)
            scale = BETA + (1.0 - BETA) * _vrsqrt(sv)
            for k in range(KD):
                rows[pl.ds(off + k * L, L)] = regs[k] * scale
        return carry

    lax.fori_loop(0, BPT // L, group, 0)
    pltpu.async_copy(rows, out_hbm.at[widx], ssem).wait()


_sc_update = _mpmd._mpmd_map(
    [(_mesh, _upd_body)],
    (jax.ShapeDtypeStruct((D * V,), jnp.float32),),
    input_output_aliases={2: 0},
    compiler_params=_sc_params,
    scratch_types=[
        pltpu.VMEM((BPT,), jnp.int32),
        pltpu.VMEM((BPT * D,), jnp.int32),
        pltpu.VMEM((BPT * D,), jnp.float32),
        pltpu.SemaphoreType.DMA,
    ],
)


BB = 512
_INV_SQRT_D = 1.0 / math.sqrt(D)
_INV_SQRT_2 = 1.0 / math.sqrt(2.0)


def _ln(x, g, b, eps=1e-5):
    mu = jnp.mean(x, axis=-1, keepdims=True)
    xc = x - mu
    var = jnp.mean(xc * xc, axis=-1, keepdims=True)
    return xc * lax.rsqrt(var + eps) * g + b


def _tc_body(x_ref, v_ref, z_ref, wq, bq, wk, bk, wv, bv, wo, bo,
             w1, b1, w2, b2, g1, be1, g2, be2, out_ref):
    xb = x_ref[...]
    vb = v_ref[...]
    zb = z_ref[...]
    S = jnp.concatenate([xb, vb, zb], axis=0)
    f32 = jnp.float32
    Q = jnp.dot(S, wq[...], preferred_element_type=f32) + bq[...]
    K = jnp.dot(S, wk[...], preferred_element_type=f32) + bk[...]
    Vv = jnp.dot(S, wv[...], preferred_element_type=f32) + bv[...]
    q = [Q[i * BB:(i + 1) * BB] for i in range(3)]
    k = [K[i * BB:(i + 1) * BB] for i in range(3)]
    v = [Vv[i * BB:(i + 1) * BB] for i in range(3)]
    ao = []
    for i in range(3):
        sc = [jnp.sum(q[i] * k[j], axis=-1, keepdims=True) * _INV_SQRT_D
              for j in range(3)]
        m = jnp.maximum(jnp.maximum(sc[0], sc[1]), sc[2])
        e = [jnp.exp(s - m) for s in sc]
        den = e[0] + e[1] + e[2]
        ao.append((e[0] * v[0] + e[1] * v[1] + e[2] * v[2]) / den)
    AO = jnp.concatenate(ao, axis=0)
    P = jnp.dot(AO, wo[...], preferred_element_type=f32) + bo[...]
    H1 = _ln(S + P, g1[...], be1[...])
    U = jnp.dot(H1, w1[...], preferred_element_type=f32) + b1[...]
    U = 0.5 * U * (1.0 + lax.erf(U * _INV_SQRT_2))
    FF = jnp.dot(U, w2[...], preferred_element_type=f32) + b2[...]
    H2 = _ln(H1 + FF, g2[...], be2[...])
    out_ref[...] = (H2[0:BB] + H2[BB:2 * BB] + H2[2 * BB:3 * BB]) * (1.0 / 3.0)


def _tc_transformer(x, v, z, wq, bq, wk, bk, wv, bv, wo, bo,
                    w1, b1, w2, b2, g1, be1, g2, be2):
    bspec = pl.BlockSpec((BB, D), lambda i: (i, 0))
    full = lambda r, c: pl.BlockSpec((r, c), lambda i: (0, 0))
    return pl.pallas_call(
        _tc_body,
        grid=(B // BB,),
        in_specs=[
            bspec, bspec, bspec,
            full(D, D), full(1, D), full(D, D), full(1, D),
            full(D, D), full(1, D), full(D, D), full(1, D),
            full(D, H), full(1, H), full(H, D), full(1, D),
            full(1, D), full(1, D), full(1, D), full(1, D),
        ],
        out_specs=bspec,
        out_shape=jax.ShapeDtypeStruct((B, D), jnp.float32),
    )(x, v, z, wq, bq, wk, bk, wv, bv, wo, bo, w1, b1, w2, b2,
      g1, be1, g2, be2)


def kernel(x, ind, aux_table, proto_table, Wq, bq, Wk, bk, Wv, bv, Wo, bo,
           W1, b1, W2, b2, ln1_g, ln1_b, ln2_g, ln2_b):
    ind32 = ind.astype(jnp.int32)
    aux_t = aux_table.T.reshape(D * V)
    proto_t = proto_table.T.reshape(D * V)
    v_f, z_f = _sc_gather(ind32, aux_t, proto_t)
    v = v_f.reshape(B, D)
    z = z_f.reshape(B, D)
    enriched = _tc_transformer(
        x, v, z, Wq, bq.reshape(1, D), Wk, bk.reshape(1, D),
        Wv, bv.reshape(1, D), Wo, bo.reshape(1, D),
        W1, b1.reshape(1, H), W2, b2.reshape(1, D),
        ln1_g.reshape(1, D), ln1_b.reshape(1, D),
        ln2_g.reshape(1, D), ln2_b.reshape(1, D))
    (new_t,) = _sc_update(ind32, z_f, proto_t)
    new_proto = new_t.reshape(D, V).T
    return enriched, new_proto


# speedup vs baseline: 9.1882x; 9.1882x over previous
import functools
import math

import jax
import jax.numpy as jnp
from jax import lax
from jax.experimental import pallas as pl
from jax.experimental.pallas import tpu as pltpu
from jax.experimental.pallas import tpu_sc as plsc
from jax._src.pallas import mpmd as _mpmd

B = 16384
D = 64
V = 1000000
H = 1024
BETA = 0.95

NC = 2
NS = 16
NW = NC * NS
L = 16

VP = V // 2
PPT = VP // NW
BPT = B // NW
CH = 256

_mesh = plsc.VectorSubcoreMesh(core_axis_name="c", subcore_axis_name="s")
_sc_params = pltpu.CompilerParams(use_tc_tiling_on_sc=True,
                                 needs_layout_passes=False)


@functools.partial(
    pl.kernel,
    out_type=(jax.ShapeDtypeStruct((B, D), jnp.float32),
              jax.ShapeDtypeStruct((B, D), jnp.float32)),
    mesh=_mesh,
    compiler_params=_sc_params,
    scratch_types=[
        pltpu.VMEM((BPT + L,), jnp.int32),
        pltpu.VMEM((BPT,), jnp.int32),
        pltpu.VMEM((CH, 2 * D), jnp.float32),
        pltpu.VMEM((CH, D), jnp.float32),
        pltpu.SemaphoreType.DMA,
    ],
)
def _sc_gather(ind_hbm, aux_hbm, proto_hbm, v_out, z_out, idx_v, pidx,
               pairs, rows, sem):
    wid = lax.axis_index("s") * NC + lax.axis_index("c")
    base = wid * BPT
    pltpu.sync_copy(ind_hbm.at[pl.ds(base, BPT)], idx_v.at[pl.ds(0, BPT)])

    def fill(g, carry):
        pidx[pl.ds(g * L, L)] = idx_v[pl.ds(g * L, L)] >> 1
        return carry

    lax.fori_loop(0, BPT // L, fill, 0)

    for tab, out in ((aux_hbm, v_out), (proto_hbm, z_out)):
        for h in range(BPT // CH):
            pltpu.async_copy(tab.at[pidx.at[pl.ds(h * CH, CH)]], pairs,
                             sem).wait()

            def ext(r, carry):
                iv = idx_v[pl.ds(h * CH + r, L)]
                half = (iv[0] & 1) * D
                for k in range(D // L):
                    rows[r, pl.ds(k * L, L)] = pairs[r, pl.ds(half + k * L, L)]
                return carry

            lax.fori_loop(0, CH, ext, 0)
            pltpu.sync_copy(rows, out.at[pl.ds(base + h * CH, CH)])


def _vrsqrt(s):
    i = plsc.bitcast(s, jnp.int32)
    y = plsc.bitcast(jnp.int32(0x5F3759DF) - (i >> 1), jnp.float32)
    for _ in range(4):
        y = y * (1.5 - 0.5 * s * y * y)
    return y


def _upd_body(ind_hbm, zp_hbm, proto_hbm, out_hbm, ind_v, selidx, selpos,
              pidx_c, zpid_c, outp, zp, gsem, zsem, ssem):
    wid = lax.axis_index("s") * NC + lax.axis_index("c")
    lo = wid * PPT
    hi = lo + PPT
    pltpu.sync_copy(ind_hbm, ind_v)

    pos0 = lax.iota(jnp.int32, L)

    for parity in range(2):
        sel_i = selidx
        sel_p = selpos

        def scan(i, cnt):
            iv = ind_v[pl.ds(i * L, L)]
            pv = iv >> 1
            m = (pv >= lo) & (pv < hi) & ((iv & 1) == parity)
            cm = plsc.cumsum(m.astype(jnp.int32))
            plsc.store_scatter(sel_i, [cnt + cm - 1], iv, mask=m)
            plsc.store_scatter(sel_p, [cnt + cm - 1], pos0 + i * L, mask=m)
            return cnt + jnp.sum(m.astype(jnp.int32))

        cnt = lax.fori_loop(0, B // L, scan, jnp.int32(0))

        i0 = jnp.full((L,), sel_i[pl.ds(0, L)][0], dtype=jnp.int32)
        p0 = jnp.full((L,), sel_p[pl.ds(0, L)][0], dtype=jnp.int32)
        for t in range(CH // L):
            sel_i[pl.ds(cnt + t * L, L)] = i0
            sel_p[pl.ds(cnt + t * L, L)] = p0

        def chunk(c, carry):
            for t in range(CH // L):
                sl = pl.ds(c * CH + t * L, L)
                pidx_c[pl.ds(t * L, L)] = sel_i[sl] >> 1
                zpid_c[pl.ds(t * L, L)] = sel_p[sl] >> 1
            cg = pltpu.async_copy(out_hbm.at[pidx_c], outp, gsem)
            cz = pltpu.async_copy(zp_hbm.at[zpid_c], zp, zsem)
            cg.wait()
            cz.wait()

            def row(r, carry2):
                zhalf = (sel_p[pl.ds(c * CH + r, L)][0] & 1) * D
                regs = [zp[r, pl.ds(zhalf + k * L, L)] for k in range(D // L)]
                t2 = regs[0] * regs[0]
                for q in regs[1:]:
                    t2 = t2 + q * q
                s = jnp.maximum(jnp.sum(t2), 1e-24)
                sv = jnp.full((L,), s, dtype=jnp.float32)
                scale = BETA + (1.0 - BETA) * _vrsqrt(sv)
                for k in range(D // L):
                    outp[r, pl.ds(parity * D + k * L, L)] = regs[k] * scale
                return carry2

            lax.fori_loop(0, CH, row, 0)
            pltpu.async_copy(outp, out_hbm.at[pidx_c], ssem).wait()
            return carry

        lax.fori_loop(0, (cnt + CH - 1) // CH, chunk, 0)


_sc_update = _mpmd._mpmd_map(
    [(_mesh, _upd_body)],
    (jax.ShapeDtypeStruct((VP, 2 * D), jnp.float32),),
    input_output_aliases={2: 0},
    compiler_params=_sc_params,
    scratch_types=[
        pltpu.VMEM((B,), jnp.int32),
        pltpu.VMEM((B + 2 * CH,), jnp.int32),
        pltpu.VMEM((B + 2 * CH,), jnp.int32),
        pltpu.VMEM((CH,), jnp.int32),
        pltpu.VMEM((CH,), jnp.int32),
        pltpu.VMEM((CH, 2 * D), jnp.float32),
        pltpu.VMEM((CH, 2 * D), jnp.float32),
        pltpu.SemaphoreType.DMA,
        pltpu.SemaphoreType.DMA,
        pltpu.SemaphoreType.DMA,
    ],
)


BB = 512
_INV_SQRT_D = 1.0 / math.sqrt(D)
_INV_SQRT_2 = 1.0 / math.sqrt(2.0)


def _ln(x, g, b, eps=1e-5):
    mu = jnp.mean(x, axis=-1, keepdims=True)
    xc = x - mu
    var = jnp.mean(xc * xc, axis=-1, keepdims=True)
    return xc * lax.rsqrt(var + eps) * g + b


def _tc_body(x_ref, v_ref, z_ref, wq, bq, wk, bk, wv, bv, wo, bo,
             w1, b1, w2, b2, g1, be1, g2, be2, out_ref):
    xb = x_ref[...]
    vb = v_ref[...]
    zb = z_ref[...]
    S = jnp.concatenate([xb, vb, zb], axis=0)
    f32 = jnp.float32
    Q = jnp.dot(S, wq[...], preferred_element_type=f32) + bq[...]
    K = jnp.dot(S, wk[...], preferred_element_type=f32) + bk[...]
    Vv = jnp.dot(S, wv[...], preferred_element_type=f32) + bv[...]
    q = [Q[i * BB:(i + 1) * BB] for i in range(3)]
    k = [K[i * BB:(i + 1) * BB] for i in range(3)]
    v = [Vv[i * BB:(i + 1) * BB] for i in range(3)]
    ao = []
    for i in range(3):
        sc = [jnp.sum(q[i] * k[j], axis=-1, keepdims=True) * _INV_SQRT_D
              for j in range(3)]
        m = jnp.maximum(jnp.maximum(sc[0], sc[1]), sc[2])
        e = [jnp.exp(s - m) for s in sc]
        den = e[0] + e[1] + e[2]
        ao.append((e[0] * v[0] + e[1] * v[1] + e[2] * v[2]) / den)
    AO = jnp.concatenate(ao, axis=0)
    P = jnp.dot(AO, wo[...], preferred_element_type=f32) + bo[...]
    H1 = _ln(S + P, g1[...], be1[...])
    U = jnp.dot(H1, w1[...], preferred_element_type=f32) + b1[...]
    U = 0.5 * U * (1.0 + lax.erf(U * _INV_SQRT_2))
    FF = jnp.dot(U, w2[...], preferred_element_type=f32) + b2[...]
    H2 = _ln(H1 + FF, g2[...], be2[...])
    out_ref[...] = (H2[0:BB] + H2[BB:2 * BB] + H2[2 * BB:3 * BB]) * (1.0 / 3.0)


def _tc_transformer(x, v, z, wq, bq, wk, bk, wv, bv, wo, bo,
                    w1, b1, w2, b2, g1, be1, g2, be2):
    bspec = pl.BlockSpec((BB, D), lambda i: (i, 0))
    full = lambda r, c: pl.BlockSpec((r, c), lambda i: (0, 0))
    return pl.pallas_call(
        _tc_body,
        grid=(B // BB,),
        in_specs=[
            bspec, bspec, bspec,
            full(D, D), full(1, D), full(D, D), full(1, D),
            full(D, D), full(1, D), full(D, D), full(1, D),
            full(D, H), full(1, H), full(H, D), full(1, D),
            full(1, D), full(1, D), full(1, D), full(1, D),
        ],
        out_specs=bspec,
        out_shape=jax.ShapeDtypeStruct((B, D), jnp.float32),
    )(x, v, z, wq, bq, wk, bk, wv, bv, wo, bo, w1, b1, w2, b2,
      g1, be1, g2, be2)


def kernel(x, ind, aux_table, proto_table, Wq, bq, Wk, bk, Wv, bv, Wo, bo,
           W1, b1, W2, b2, ln1_g, ln1_b, ln2_g, ln2_b):
    ind32 = ind.astype(jnp.int32)
    aux_p = aux_table.reshape(VP, 2 * D)
    proto_p = proto_table.reshape(VP, 2 * D)
    v, z = _sc_gather(ind32, aux_p, proto_p)
    enriched = _tc_transformer(
        x, v, z, Wq, bq.reshape(1, D), Wk, bk.reshape(1, D),
        Wv, bv.reshape(1, D), Wo, bo.reshape(1, D),
        W1, b1.reshape(1, H), W2, b2.reshape(1, D),
        ln1_g.reshape(1, D), ln1_b.reshape(1, D),
        ln2_g.reshape(1, D), ln2_b.reshape(1, D))
    zp = z.reshape(B // 2, 2 * D)
    (new_p,) = _sc_update(ind32, zp, proto_p)
    new_proto = new_p.reshape(V, D)
    return enriched, new_proto


# speedup vs baseline: 14.5563x; 1.5842x over previous
import functools
import math

import jax
import jax.numpy as jnp
from jax import lax
from jax.experimental import pallas as pl
from jax.experimental.pallas import tpu as pltpu
from jax.experimental.pallas import tpu_sc as plsc
from jax._src.pallas import mpmd as _mpmd

B = 16384
D = 64
V = 1000000
H = 1024
BETA = 0.95

NC = 2
NS = 16
NW = NC * NS
L = 16

BPT = B // NW

_mesh = plsc.VectorSubcoreMesh(core_axis_name="c", subcore_axis_name="s")
_sc_params = pltpu.CompilerParams(use_tc_tiling_on_sc=True,
                                 needs_layout_passes=False)


@functools.partial(
    pl.kernel,
    out_type=(jax.ShapeDtypeStruct((B, D), jnp.float32),
              jax.ShapeDtypeStruct((B, D), jnp.float32)),
    mesh=_mesh,
    compiler_params=_sc_params,
    scratch_types=[
        pltpu.VMEM((BPT + L,), jnp.int32),
        pltpu.VMEM((BPT, D), jnp.float32),
        pltpu.SemaphoreType.DMA,
    ],
)
def _sc_gather(ind_hbm, aux_hbm, proto_hbm, v_out, z_out, idx_v, rows, sem):
    wid = lax.axis_index("s") * NC + lax.axis_index("c")
    base = wid * BPT
    pltpu.sync_copy(ind_hbm.at[pl.ds(base, BPT)], idx_v.at[pl.ds(0, BPT)])

    for tab, out in ((aux_hbm, v_out), (proto_hbm, z_out)):
        def g16(g, carry):
            iv = idx_v[pl.ds(g * L, L)]
            copies = [
                pltpu.async_copy(tab.at[pl.ds(iv[r], 1)],
                                 rows.at[pl.ds(g * L + r, 1)], sem)
                for r in range(L)
            ]
            for c in copies:
                c.wait()
            return carry

        lax.fori_loop(0, BPT // L, g16, 0)
        pltpu.sync_copy(rows, out.at[pl.ds(base, BPT)])


def _vrsqrt(s):
    i = plsc.bitcast(s, jnp.int32)
    y = plsc.bitcast(jnp.int32(0x5F3759DF) - (i >> 1), jnp.float32)
    for _ in range(4):
        y = y * (1.5 - 0.5 * s * y * y)
    return y


def _upd_body(ind_hbm, z_hbm, proto_hbm, out_hbm, idx_v, zz, ssem):
    wid = lax.axis_index("s") * NC + lax.axis_index("c")
    base = wid * BPT
    pltpu.sync_copy(ind_hbm.at[pl.ds(base, BPT)], idx_v.at[pl.ds(0, BPT)])
    pltpu.sync_copy(z_hbm.at[pl.ds(base, BPT)], zz)

    def g16(g, carry):
        iv = idx_v[pl.ds(g * L, L)]
        for r in range(L):
            row = g * L + r
            regs = [zz[row, pl.ds(k * L, L)] for k in range(D // L)]
            t = regs[0] * regs[0]
            for q in regs[1:]:
                t = t + q * q
            s = jnp.maximum(jnp.sum(t), 1e-24)
            sv = jnp.full((L,), s, dtype=jnp.float32)
            scale = BETA + (1.0 - BETA) * _vrsqrt(sv)
            for k in range(D // L):
                zz[row, pl.ds(k * L, L)] = regs[k] * scale
        copies = [
            pltpu.async_copy(zz.at[pl.ds(g * L + r, 1)],
                             out_hbm.at[pl.ds(iv[r], 1)], ssem)
            for r in range(L)
        ]
        for c in copies:
            c.wait()
        return carry

    lax.fori_loop(0, BPT // L, g16, 0)


_sc_update = _mpmd._mpmd_map(
    [(_mesh, _upd_body)],
    (jax.ShapeDtypeStruct((V, D), jnp.float32),),
    input_output_aliases={2: 0},
    compiler_params=_sc_params,
    scratch_types=[
        pltpu.VMEM((BPT + L,), jnp.int32),
        pltpu.VMEM((BPT, D), jnp.float32),
        pltpu.SemaphoreType.DMA,
    ],
)


BB = 512
_INV_SQRT_D = 1.0 / math.sqrt(D)
_INV_SQRT_2 = 1.0 / math.sqrt(2.0)


def _ln(x, g, b, eps=1e-5):
    mu = jnp.mean(x, axis=-1, keepdims=True)
    xc = x - mu
    var = jnp.mean(xc * xc, axis=-1, keepdims=True)
    return xc * lax.rsqrt(var + eps) * g + b


def _tc_body(x_ref, v_ref, z_ref, wq, bq, wk, bk, wv, bv, wo, bo,
             w1, b1, w2, b2, g1, be1, g2, be2, out_ref):
    xb = x_ref[...]
    vb = v_ref[...]
    zb = z_ref[...]
    S = jnp.concatenate([xb, vb, zb], axis=0)
    f32 = jnp.float32
    Q = jnp.dot(S, wq[...], preferred_element_type=f32) + bq[...]
    K = jnp.dot(S, wk[...], preferred_element_type=f32) + bk[...]
    Vv = jnp.dot(S, wv[...], preferred_element_type=f32) + bv[...]
    q = [Q[i * BB:(i + 1) * BB] for i in range(3)]
    k = [K[i * BB:(i + 1) * BB] for i in range(3)]
    v = [Vv[i * BB:(i + 1) * BB] for i in range(3)]
    ao = []
    for i in range(3):
        sc = [jnp.sum(q[i] * k[j], axis=-1, keepdims=True) * _INV_SQRT_D
              for j in range(3)]
        m = jnp.maximum(jnp.maximum(sc[0], sc[1]), sc[2])
        e = [jnp.exp(s - m) for s in sc]
        den = e[0] + e[1] + e[2]
        ao.append((e[0] * v[0] + e[1] * v[1] + e[2] * v[2]) / den)
    AO = jnp.concatenate(ao, axis=0)
    P = jnp.dot(AO, wo[...], preferred_element_type=f32) + bo[...]
    H1 = _ln(S + P, g1[...], be1[...])
    U = jnp.dot(H1, w1[...], preferred_element_type=f32) + b1[...]
    U = 0.5 * U * (1.0 + lax.erf(U * _INV_SQRT_2))
    FF = jnp.dot(U, w2[...], preferred_element_type=f32) + b2[...]
    H2 = _ln(H1 + FF, g2[...], be2[...])
    out_ref[...] = (H2[0:BB] + H2[BB:2 * BB] + H2[2 * BB:3 * BB]) * (1.0 / 3.0)


def _tc_transformer(x, v, z, wq, bq, wk, bk, wv, bv, wo, bo,
                    w1, b1, w2, b2, g1, be1, g2, be2):
    bspec = pl.BlockSpec((BB, D), lambda i: (i, 0))
    full = lambda r, c: pl.BlockSpec((r, c), lambda i: (0, 0))
    return pl.pallas_call(
        _tc_body,
        grid=(B // BB,),
        in_specs=[
            bspec, bspec, bspec,
            full(D, D), full(1, D), full(D, D), full(1, D),
            full(D, D), full(1, D), full(D, D), full(1, D),
            full(D, H), full(1, H), full(H, D), full(1, D),
            full(1, D), full(1, D), full(1, D), full(1, D),
        ],
        out_specs=bspec,
        out_shape=jax.ShapeDtypeStruct((B, D), jnp.float32),
    )(x, v, z, wq, bq, wk, bk, wv, bv, wo, bo, w1, b1, w2, b2,
      g1, be1, g2, be2)


def kernel(x, ind, aux_table, proto_table, Wq, bq, Wk, bk, Wv, bv, Wo, bo,
           W1, b1, W2, b2, ln1_g, ln1_b, ln2_g, ln2_b):
    ind32 = ind.astype(jnp.int32)
    v, z = _sc_gather(ind32, aux_table, proto_table)
    enriched = _tc_transformer(
        x, v, z, Wq, bq.reshape(1, D), Wk, bk.reshape(1, D),
        Wv, bv.reshape(1, D), Wo, bo.reshape(1, D),
        W1, b1.reshape(1, H), W2, b2.reshape(1, D),
        ln1_g.reshape(1, D), ln1_b.reshape(1, D),
        ln2_g.reshape(1, D), ln2_b.reshape(1, D))
    (new_proto,) = _sc_update(ind32, z, proto_table)
    return enriched, new_proto
